# baseline (speedup 1.0000x reference)
import functools

import jax
import jax.numpy as jnp
from jax.experimental import pallas as pl
from jax.experimental.pallas import tpu as pltpu

_NUM_STAGES = 4
_K = 1024
_D = 256
_BLK = 1024


def _rvq_kernel(x_ref, cb_ref, quant_ref, codes_ref, loss_ref,
                hi_ref, mid_ref, lo_ref, cn_ref):
    i = pl.program_id(0)

    @pl.when(i == 0)
    def _prep():
        cb = cb_ref[...]
        hi = cb.astype(jnp.bfloat16)
        r1 = cb - hi.astype(jnp.float32)
        mid = r1.astype(jnp.bfloat16)
        lo = (r1 - mid.astype(jnp.float32)).astype(jnp.bfloat16)
        hi_ref[...] = hi
        mid_ref[...] = mid
        lo_ref[...] = lo
        cn_ref[...] = jnp.sum(cb * cb, axis=2)

    iota_row = jax.lax.broadcasted_iota(
        jnp.int32, (1, _K), 1).astype(jnp.float32)

    def _stage(r, s):
        a = jnp.sum(r * r, axis=1, keepdims=True)
        b = jax.lax.dot_general(
            r, cb_ref[s], (((1,), (1,)), ((), ())),
            preferred_element_type=jnp.float32)
        c = cn_ref[s][None, :]
        dists = a - 2.0 * b + c
        m = jnp.min(dists, axis=1, keepdims=True)
        masked = jnp.where(dists == m, iota_row, jnp.float32(_K))
        idxf = jnp.min(masked, axis=1, keepdims=True)
        onehot = (iota_row == idxf).astype(jnp.bfloat16)

        def _oh_dot(mat):
            return jax.lax.dot_general(
                onehot, mat, (((1,), (0,)), ((), ())),
                preferred_element_type=jnp.float32)

        q = (_oh_dot(hi_ref[s]) + _oh_dot(mid_ref[s])) + _oh_dot(lo_ref[s])
        return q, idxf, m

    _NH = 4
    _H = _BLK // _NH
    xs = [x_ref[h * _H:(h + 1) * _H, :] for h in range(_NH)]
    rs = list(xs)
    qsums = [jnp.zeros_like(xs[0]) for _ in range(_NH)]
    loss = jnp.float32(0.0)
    for s in range(_NUM_STAGES):
        for h in range(_NH):
            q, idxf, m = _stage(rs[h], s)
            loss = loss + jnp.sum(m)
            codes_ref[h * _H:(h + 1) * _H, s:s + 1] = idxf.astype(jnp.int32)
            qsums[h] = qsums[h] + q
            rs[h] = rs[h] - q
    for h in range(_NH):
        quant_ref[h * _H:(h + 1) * _H, :] = xs[h] + (qsums[h] - xs[h])

    loss2d = loss.reshape(1, 1)

    @pl.when(i == 0)
    def _init():
        loss_ref[...] = loss2d

    @pl.when(i != 0)
    def _acc():
        loss_ref[...] += loss2d


@functools.partial(jax.jit, static_argnames=())
def kernel(inputs, codebooks):
    B, N, D = inputs.shape
    tokens = B * N
    flat = inputs.reshape(tokens, D)
    grid = tokens // _BLK
    quant, codes, loss = pl.pallas_call(
        _rvq_kernel,
        grid=(grid,),
        in_specs=[
            pl.BlockSpec((_BLK, D), lambda i: (i, 0)),
            pl.BlockSpec((_NUM_STAGES, _K, D), lambda i: (0, 0, 0)),
        ],
        out_specs=[
            pl.BlockSpec((_BLK, D), lambda i: (i, 0)),
            pl.BlockSpec((_BLK, _NUM_STAGES), lambda i: (i, 0)),
            pl.BlockSpec((1, 1), lambda i: (0, 0)),
        ],
        out_shape=[
            jax.ShapeDtypeStruct((tokens, D), jnp.float32),
            jax.ShapeDtypeStruct((tokens, _NUM_STAGES), jnp.int32),
            jax.ShapeDtypeStruct((1, 1), jnp.float32),
        ],
        scratch_shapes=[
            pltpu.VMEM((_NUM_STAGES, _K, _D), jnp.bfloat16),
            pltpu.VMEM((_NUM_STAGES, _K, _D), jnp.bfloat16),
            pltpu.VMEM((_NUM_STAGES, _K, _D), jnp.bfloat16),
            pltpu.VMEM((_NUM_STAGES, _K), jnp.float32),
        ],
    )(flat, codebooks)
    scale = (1.0 + 0.25) / jnp.float32(tokens * D)
    total_loss = loss[0, 0] * scale
    quantized = quant.reshape(B, N, D)
    codes = codes.T.reshape(_NUM_STAGES, B, N)
    return quantized, total_loss, codes


# speedup vs baseline: 1.1038x; 1.1038x over previous
import functools

import jax
import jax.numpy as jnp
from jax.experimental import pallas as pl
from jax.experimental.pallas import tpu as pltpu

_NUM_STAGES = 4
_K = 1024
_D = 256
_BLK = 2048


def _rvq_kernel(x_ref, cb_ref, quant_ref, codes_ref, loss_ref,
                hi_ref, mid_ref, lo_ref, cn_ref):
    i = pl.program_id(0)

    @pl.when(i == 0)
    def _prep():
        cb = cb_ref[...]
        hi = cb.astype(jnp.bfloat16)
        r1 = cb - hi.astype(jnp.float32)
        mid = r1.astype(jnp.bfloat16)
        lo = (r1 - mid.astype(jnp.float32)).astype(jnp.bfloat16)
        hi_ref[...] = hi
        mid_ref[...] = mid
        lo_ref[...] = lo
        cn_ref[...] = jnp.sum(cb * cb, axis=2)

    iota_row = jax.lax.broadcasted_iota(
        jnp.int32, (1, _K), 1).astype(jnp.float32)

    def _stage(r, s):
        a = jnp.sum(r * r, axis=1, keepdims=True)
        b = jax.lax.dot_general(
            r, cb_ref[s], (((1,), (1,)), ((), ())),
            preferred_element_type=jnp.float32)
        c = cn_ref[s][None, :]
        dists = a - 2.0 * b + c
        m = jnp.min(dists, axis=1, keepdims=True)
        masked = jnp.where(dists == m, iota_row, jnp.float32(_K))
        idxf = jnp.min(masked, axis=1, keepdims=True)
        onehot = (iota_row == idxf).astype(jnp.bfloat16)

        def _oh_dot(mat):
            return jax.lax.dot_general(
                onehot, mat, (((1,), (0,)), ((), ())),
                preferred_element_type=jnp.float32)

        q = (_oh_dot(hi_ref[s]) + _oh_dot(mid_ref[s])) + _oh_dot(lo_ref[s])
        return q, idxf, m

    _NH = 2
    _H = _BLK // _NH
    xs = [x_ref[h * _H:(h + 1) * _H, :] for h in range(_NH)]
    rs = list(xs)
    qsums = [jnp.zeros_like(xs[0]) for _ in range(_NH)]
    loss = jnp.float32(0.0)
    for s in range(_NUM_STAGES):
        for h in range(_NH):
            q, idxf, m = _stage(rs[h], s)
            loss = loss + jnp.sum(m)
            codes_ref[h * _H:(h + 1) * _H, s:s + 1] = idxf.astype(jnp.int32)
            qsums[h] = qsums[h] + q
            rs[h] = rs[h] - q
    for h in range(_NH):
        quant_ref[h * _H:(h + 1) * _H, :] = xs[h] + (qsums[h] - xs[h])

    loss2d = loss.reshape(1, 1)

    @pl.when(i == 0)
    def _init():
        loss_ref[...] = loss2d

    @pl.when(i != 0)
    def _acc():
        loss_ref[...] += loss2d


@functools.partial(jax.jit, static_argnames=())
def kernel(inputs, codebooks):
    B, N, D = inputs.shape
    tokens = B * N
    flat = inputs.reshape(tokens, D)
    grid = tokens // _BLK
    quant, codes, loss = pl.pallas_call(
        _rvq_kernel,
        grid=(grid,),
        in_specs=[
            pl.BlockSpec((_BLK, D), lambda i: (i, 0)),
            pl.BlockSpec((_NUM_STAGES, _K, D), lambda i: (0, 0, 0)),
        ],
        out_specs=[
            pl.BlockSpec((_BLK, D), lambda i: (i, 0)),
            pl.BlockSpec((_BLK, _NUM_STAGES), lambda i: (i, 0)),
            pl.BlockSpec((1, 1), lambda i: (0, 0)),
        ],
        out_shape=[
            jax.ShapeDtypeStruct((tokens, D), jnp.float32),
            jax.ShapeDtypeStruct((tokens, _NUM_STAGES), jnp.int32),
            jax.ShapeDtypeStruct((1, 1), jnp.float32),
        ],
        scratch_shapes=[
            pltpu.VMEM((_NUM_STAGES, _K, _D), jnp.bfloat16),
            pltpu.VMEM((_NUM_STAGES, _K, _D), jnp.bfloat16),
            pltpu.VMEM((_NUM_STAGES, _K, _D), jnp.bfloat16),
            pltpu.VMEM((_NUM_STAGES, _K), jnp.float32),
        ],
    )(flat, codebooks)
    scale = (1.0 + 0.25) / jnp.float32(tokens * D)
    total_loss = loss[0, 0] * scale
    quantized = quant.reshape(B, N, D)
    codes = codes.T.reshape(_NUM_STAGES, B, N)
    return quantized, total_loss, codes
